# Initial kernel scaffold; baseline (speedup 1.0000x reference)
import jax
import jax.numpy as jnp
from jax.experimental import pallas as pl


def kernel(edge_index, node2sub, max_nodes, W1, b1, W2, b2, W3, b3, W4, b4, Wf0, bf0, Wf1, bf1):
    raise NotImplementedError("write your pallas kernel here")


# speedup vs baseline: 7.7734x; 7.7734x over previous
import functools

import jax
import jax.numpy as jnp
from jax import lax
from jax.experimental import pallas as pl
from jax.experimental.pallas import tpu as pltpu
from jax.experimental.pallas import tpu_sc as plsc

N_NODES = 50000
N_EDGES = 800000
HIDDEN = 64
N_SUB = 2048

NPAD = 51200
HALF = 25600
EPAD = 819200
CH = 128

ACC_ROWS = 26624
DUMMY_BASE = HALF
POOL_PAD = 2560

_MESH = plsc.VectorSubcoreMesh(core_axis_name="c", subcore_axis_name="s")


def _zero_1d(ref, n):
    z = jnp.zeros((16,), ref.dtype)

    def body(i, _):
        ref[pl.ds(i * 16, 16)] = z
        return 0

    lax.fori_loop(0, n // 16, body, 0)


def _sc_degree_body(dst_hbm, hist_out, dstbuf, onesbuf, zbuf, hist_sh):
    cid = lax.axis_index("c")
    sid = lax.axis_index("s")
    gwid = cid * 16 + sid

    _zero_1d(zbuf, 3200)
    pltpu.sync_copy(zbuf, hist_sh.at[pl.ds(sid * 3200, 3200)])

    def ones_body(i, _):
        onesbuf[pl.ds(i * 16, 16)] = jnp.ones((16,), jnp.float32)
        return 0

    lax.fori_loop(0, CH // 16, ones_body, 0)
    plsc.subcore_barrier()

    ebase = gwid * (EPAD // 32)

    def chunk(c, _):
        pltpu.sync_copy(dst_hbm.at[pl.ds(ebase + c * CH, CH)], dstbuf)
        pltpu.sync_copy(onesbuf, hist_sh.at[dstbuf], add=True)
        return 0

    lax.fori_loop(0, (EPAD // 32) // CH, chunk, 0)
    plsc.subcore_barrier()
    pltpu.sync_copy(hist_sh.at[pl.ds(sid * 3200, 3200)],
                    hist_out.at[cid, pl.ds(sid * 3200, 3200)])


def _sc_degree(dst_pad):
    return pl.kernel(
        _sc_degree_body,
        out_type=jax.ShapeDtypeStruct((2, NPAD), jnp.float32),
        mesh=_MESH,
        scratch_types=[
            pltpu.VMEM((CH,), jnp.int32),
            pltpu.VMEM((CH,), jnp.float32),
            pltpu.VMEM((3200,), jnp.float32),
            pltpu.VMEM_SHARED((NPAD,), jnp.float32),
        ],
    )(dst_pad)


def _sc_agg_body(src_hbm, dst_hbm, h1_hbm, agg_out,
                 srcbuf, dstbuf, lidxbuf, rowsbuf, acc_sh):
    cid = lax.axis_index("c")
    sid = lax.axis_index("s")

    def zrow(i, _):
        for j in range(HIDDEN // 16):
            rowsbuf[i, pl.ds(j * 16, 16)] = jnp.zeros((16,), jnp.float32)
        return 0

    lax.fori_loop(0, CH, zrow, 0)

    def zacc(k, _):
        pltpu.sync_copy(rowsbuf, acc_sh.at[pl.ds(sid * 1664 + k * CH, CH)])
        return 0

    lax.fori_loop(0, 1664 // CH, zacc, 0)
    plsc.subcore_barrier()

    nbase = cid * HALF
    ebase = sid * (EPAD // 16)

    def chunk(c, _):
        off = ebase + c * CH
        pltpu.sync_copy(src_hbm.at[pl.ds(off, CH)], srcbuf)
        pltpu.sync_copy(dst_hbm.at[pl.ds(off, CH)], dstbuf)
        pltpu.sync_copy(h1_hbm.at[srcbuf], rowsbuf)

        def lidx(j, _):
            d = dstbuf[pl.ds(j * 16, 16)]
            s = srcbuf[pl.ds(j * 16, 16)]
            ld = d - nbase
            ok = (ld >= 0) & (ld < HALF)
            dummy = DUMMY_BASE + (s & 511)
            lidxbuf[pl.ds(j * 16, 16)] = jnp.where(ok, ld, dummy)
            return 0

        lax.fori_loop(0, CH // 16, lidx, 0)
        pltpu.sync_copy(rowsbuf, acc_sh.at[lidxbuf], add=True)
        return 0

    lax.fori_loop(0, (EPAD // 16) // CH, chunk, 0)
    plsc.subcore_barrier()
    pltpu.sync_copy(acc_sh.at[pl.ds(sid * 1600, 1600)],
                    agg_out.at[cid, pl.ds(sid * 1600, 1600)])


def _sc_agg(src_pad, dst_pad, h1):
    return pl.kernel(
        _sc_agg_body,
        out_type=jax.ShapeDtypeStruct((2, HALF, HIDDEN), jnp.float32),
        mesh=_MESH,
        compiler_params=pltpu.CompilerParams(use_tc_tiling_on_sc=False),
        scratch_types=[
            pltpu.VMEM((CH,), jnp.int32),
            pltpu.VMEM((CH,), jnp.int32),
            pltpu.VMEM((CH,), jnp.int32),
            pltpu.VMEM((CH, HIDDEN), jnp.float32),
            pltpu.VMEM_SHARED((ACC_ROWS, HIDDEN), jnp.float32),
        ],
    )(src_pad, dst_pad, h1)


def _sc_pool_body(g_hbm, n2s_hbm, pool_out, gbuf, idxbuf, zbuf, pool_sh):
    cid = lax.axis_index("c")
    sid = lax.axis_index("s")

    _zero_1d(zbuf, 160)
    pltpu.sync_copy(zbuf, pool_sh.at[pl.ds(sid * 160, 160)])
    plsc.subcore_barrier()

    nbase = cid * HALF + sid * 1600

    def chunk(c, _):
        off = nbase + c * 80
        pltpu.sync_copy(g_hbm.at[pl.ds(off, 80)], gbuf)
        pltpu.sync_copy(n2s_hbm.at[pl.ds(off, 80)], idxbuf)
        pltpu.sync_copy(gbuf, pool_sh.at[idxbuf], add=True)
        return 0

    lax.fori_loop(0, 1600 // 80, chunk, 0)
    plsc.subcore_barrier()
    pltpu.sync_copy(pool_sh.at[pl.ds(sid * 128, 128)],
                    pool_out.at[cid, pl.ds(sid * 128, 128)])


def _sc_pool(g, n2s_pad):
    return pl.kernel(
        _sc_pool_body,
        out_type=jax.ShapeDtypeStruct((2, N_SUB), jnp.float32),
        mesh=_MESH,
        scratch_types=[
            pltpu.VMEM((80,), jnp.float32),
            pltpu.VMEM((80,), jnp.int32),
            pltpu.VMEM((160,), jnp.float32),
            pltpu.VMEM_SHARED((POOL_PAD,), jnp.float32),
        ],
    )(g, n2s_pad)


_INV_SQRT2 = 0.7071067811865476


def _gelu(x):
    return 0.5 * x * (1.0 + lax.erf(x * _INV_SQRT2))


def _dot(a, b):
    return jnp.dot(a, b, preferred_element_type=jnp.float32,
                   precision=lax.Precision.HIGHEST)


_ROWS = 512


def _tc_h1_body(dp_ref, w1_ref, b1_ref, w2_ref, b2_ref, out_ref):
    d = dp_ref[0, :] + dp_ref[1, :] + 1.0
    z = d[:, None] * w1_ref[0, :][None, :] + b1_ref[0, :][None, :]
    h = _gelu(z)
    out_ref[...] = _gelu(_dot(h, w2_ref[...]) + b2_ref[0, :][None, :])


def _tc_h1(deg_partials, W1, b1, W2, b2):
    return pl.pallas_call(
        _tc_h1_body,
        grid=(NPAD // _ROWS,),
        in_specs=[
            pl.BlockSpec((2, _ROWS), lambda i: (0, i)),
            pl.BlockSpec((1, HIDDEN), lambda i: (0, 0)),
            pl.BlockSpec((1, HIDDEN), lambda i: (0, 0)),
            pl.BlockSpec((HIDDEN, HIDDEN), lambda i: (0, 0)),
            pl.BlockSpec((1, HIDDEN), lambda i: (0, 0)),
        ],
        out_specs=pl.BlockSpec((_ROWS, HIDDEN), lambda i: (i, 0)),
        out_shape=jax.ShapeDtypeStruct((NPAD, HIDDEN), jnp.float32),
    )(deg_partials, W1, b1, W2, b2)


def _tc_g_body(h1_ref, agg_ref, w3_ref, b3_ref, w4_ref, b4_ref,
               wf0_ref, bf0_ref, wf1_ref, out_ref):
    h = h1_ref[...] + agg_ref[...]
    h = _gelu(_dot(h, w3_ref[...]) + b3_ref[0, :][None, :])
    h = _gelu(_dot(h, w4_ref[...]) + b4_ref[0, :][None, :])
    t = _gelu(_dot(h, wf0_ref[...]) + bf0_ref[0, :][None, :])
    out_ref[...] = _dot(t, wf1_ref[...])


def _tc_g(h1, agg, W3, b3, W4, b4, Wf0, bf0, Wf1):
    full = lambda i: (0, 0)
    return pl.pallas_call(
        _tc_g_body,
        grid=(NPAD // _ROWS,),
        in_specs=[
            pl.BlockSpec((_ROWS, HIDDEN), lambda i: (i, 0)),
            pl.BlockSpec((_ROWS, HIDDEN), lambda i: (i, 0)),
            pl.BlockSpec((HIDDEN, HIDDEN), full),
            pl.BlockSpec((1, HIDDEN), full),
            pl.BlockSpec((HIDDEN, HIDDEN), full),
            pl.BlockSpec((1, HIDDEN), full),
            pl.BlockSpec((HIDDEN, HIDDEN), full),
            pl.BlockSpec((1, HIDDEN), full),
            pl.BlockSpec((HIDDEN, 1), full),
        ],
        out_specs=pl.BlockSpec((_ROWS, 1), lambda i: (i, 0)),
        out_shape=jax.ShapeDtypeStruct((NPAD, 1), jnp.float32),
    )(h1, agg, W3, b3, W4, b4, Wf0, bf0, Wf1)


def _tc_final_body(pp_ref, bf1_ref, out_ref):
    out_ref[...] = pp_ref[0:1, :] + pp_ref[1:2, :] + bf1_ref[0, 0]


def _tc_final(pool_partials, bf1):
    return pl.pallas_call(
        _tc_final_body,
        in_specs=[
            pl.BlockSpec((2, N_SUB), lambda: (0, 0)),
            pl.BlockSpec((1, 1), lambda: (0, 0)),
        ],
        out_specs=pl.BlockSpec((1, N_SUB), lambda: (0, 0)),
        out_shape=jax.ShapeDtypeStruct((1, N_SUB), jnp.float32),
    )(pool_partials, bf1.reshape(1, 1))


def kernel(edge_index, node2sub, max_nodes, W1, b1, W2, b2, W3, b3, W4, b4,
           Wf0, bf0, Wf1, bf1):
    src = edge_index[0].astype(jnp.int32)
    dst = edge_index[1].astype(jnp.int32)
    npad_e = EPAD - N_EDGES
    padi = jnp.arange(npad_e, dtype=jnp.int32)
    src_p = jnp.concatenate([src, padi % N_NODES])
    dst_p = jnp.concatenate([dst, N_NODES + (padi % 1024)])
    n2s_p = jnp.concatenate([
        node2sub.astype(jnp.int32),
        N_SUB + (jnp.arange(NPAD - N_NODES, dtype=jnp.int32) % 512),
    ])

    deg_partials = _sc_degree(dst_p)
    h1 = _tc_h1(deg_partials, W1, b1.reshape(1, HIDDEN), W2,
                b2.reshape(1, HIDDEN))
    agg = _sc_agg(src_p, dst_p, h1).reshape(NPAD, HIDDEN)
    g = _tc_g(h1, agg, W3, b3.reshape(1, HIDDEN), W4, b4.reshape(1, HIDDEN),
              Wf0, bf0.reshape(1, HIDDEN), Wf1).reshape(NPAD)
    pool_partials = _sc_pool(g, n2s_p)
    res = _tc_final(pool_partials, bf1)
    return res.reshape(N_SUB // 256, 256)
